# baseline (speedup 1.0000x reference)
import functools

import jax
import jax.numpy as jnp
from jax import lax
from jax.experimental import pallas as pl
from jax.experimental.pallas import tpu as pltpu
from jax.experimental.pallas import tpu_sc as plsc

_COMMIT = 0.25


def _assign_body(x_ref, w_ref, idx_ref, counts_ref, esum_ref):
    x = x_ref[...]
    w = w_ref[...]
    k = w.shape[0]
    mm = lax.dot_general(x, w, (((1,), (1,)), ((), ())),
                         preferred_element_type=jnp.float32)
    xx = jnp.sum(x * x, axis=1, keepdims=True)
    ww = jnp.sum(w * w, axis=1)[None, :]
    d = (xx + ww) - 2.0 * mm
    m = jnp.min(d, axis=1, keepdims=True)
    col = lax.broadcasted_iota(jnp.int32, d.shape, 1)
    idx = jnp.min(jnp.where(d == m, col, k), axis=1)
    idx_ref[0, 0, :] = idx
    onehot = (col == idx[:, None]).astype(jnp.float32)
    cpart = jnp.sum(onehot, axis=0)[None, :]
    epart = jnp.sum(m)

    @pl.when(pl.program_id(0) == 0)
    def _init():
        counts_ref[...] = jnp.zeros_like(counts_ref)
        esum_ref[0, 0] = 0.0

    counts_ref[...] += cpart
    esum_ref[0, 0] += epart


def _epilogue_body(n_tokens, w_ref, counts_ref, esum_ref,
                   quant_ref, util_ref, compact_ref):
    w = w_ref[...]
    k = w.shape[0]
    gram = lax.dot_general(w, w, (((1,), (1,)), ((), ())),
                           preferred_element_type=jnp.float32)
    sq = jnp.sum(w * w, axis=1)
    d2 = sq[:, None] + sq[None, :] - 2.0 * gram
    d2 = jnp.maximum(d2, 0.0)
    row = lax.broadcasted_iota(jnp.int32, (k, k), 0)
    col = lax.broadcasted_iota(jnp.int32, (k, k), 1)
    mask = col > row
    dist = jnp.sqrt(jnp.where(mask, d2, 1.0))
    psum = jnp.sum(jnp.where(mask, dist, 0.0))
    n_pairs = k * (k - 1) // 2
    compact_ref[0, 0] = 2.0 * (psum / n_pairs)

    counts = counts_ref[...]
    util_ref[0, 0] = jnp.sum(jnp.abs(counts - (n_tokens / k))) / k

    d_dim = w.shape[1]
    e_latent = esum_ref[0, 0] / (n_tokens * d_dim)
    quant_ref[0, 0] = e_latent + _COMMIT * e_latent


def _st_body(x_ref, q_ref, out_ref):
    x = x_ref[...]
    q = q_ref[:, : x.shape[1]]
    out_ref[...] = x + (q - x)


def _make_sc_gather(n, d):
    nw = 32
    bpw = n // nw
    ch = 96
    nch = bpw // ch
    mesh = plsc.VectorSubcoreMesh(core_axis_name="c", subcore_axis_name="s")

    @functools.partial(
        pl.kernel, mesh=mesh,
        out_type=jax.ShapeDtypeStruct((n, d), jnp.float32),
        scratch_types=[
            pltpu.VMEM((bpw,), jnp.int32),
            pltpu.VMEM((bpw, d), jnp.float32),
            pltpu.SemaphoreType.DMA,
        ],
    )
    def gather_k(idx_hbm, w_hbm, out_hbm, idx_v, rows_v, sem):
        wid = lax.axis_index("s") * 2 + lax.axis_index("c")
        base = wid * bpw
        pltpu.sync_copy(idx_hbm.at[pl.ds(base, bpw)], idx_v)
        copies = []
        for c in range(nch):
            copies.append(pltpu.async_copy(
                w_hbm.at[idx_v.at[pl.ds(c * ch, ch)]],
                rows_v.at[pl.ds(c * ch, ch)], sem))
        for cp in copies:
            cp.wait()
        pltpu.sync_copy(rows_v, out_hbm.at[pl.ds(base, bpw)])

    return gather_k


def kernel(x, W):
    n, d = x.shape
    k = W.shape[0]
    blk = 1024
    nb = n // blk

    idx3, counts2, esum = pl.pallas_call(
        _assign_body,
        grid=(nb,),
        in_specs=[
            pl.BlockSpec((blk, d), lambda i: (i, 0)),
            pl.BlockSpec((k, d), lambda i: (0, 0)),
        ],
        out_specs=[
            pl.BlockSpec((1, 1, blk), lambda i: (i, 0, 0)),
            pl.BlockSpec((1, k), lambda i: (0, 0)),
            pl.BlockSpec((1, 1), lambda i: (0, 0),
                         memory_space=pltpu.SMEM),
        ],
        out_shape=[
            jax.ShapeDtypeStruct((nb, 1, blk), jnp.int32),
            jax.ShapeDtypeStruct((1, k), jnp.float32),
            jax.ShapeDtypeStruct((1, 1), jnp.float32),
        ],
    )(x, W)
    idx = jnp.reshape(idx3, (n,))

    w_pad = jnp.pad(W, ((0, 0), (0, 128 - d)))
    quantized = _make_sc_gather(n, 128)(idx, w_pad)

    quant2, util2, compact2 = pl.pallas_call(
        functools.partial(_epilogue_body, float(n)),
        in_specs=[
            pl.BlockSpec((k, d), lambda: (0, 0)),
            pl.BlockSpec((1, k), lambda: (0, 0)),
            pl.BlockSpec((1, 1), lambda: (0, 0), memory_space=pltpu.SMEM),
        ],
        out_specs=[
            pl.BlockSpec((1, 1), lambda: (0, 0), memory_space=pltpu.SMEM),
            pl.BlockSpec((1, 1), lambda: (0, 0), memory_space=pltpu.SMEM),
            pl.BlockSpec((1, 1), lambda: (0, 0), memory_space=pltpu.SMEM),
        ],
        out_shape=[
            jax.ShapeDtypeStruct((1, 1), jnp.float32),
            jax.ShapeDtypeStruct((1, 1), jnp.float32),
            jax.ShapeDtypeStruct((1, 1), jnp.float32),
        ],
    )(W, counts2, esum)

    quantized_st = pl.pallas_call(
        _st_body,
        grid=(nb,),
        in_specs=[
            pl.BlockSpec((blk, d), lambda i: (i, 0)),
            pl.BlockSpec((blk, 128), lambda i: (i, 0)),
        ],
        out_specs=pl.BlockSpec((blk, d), lambda i: (i, 0)),
        out_shape=jax.ShapeDtypeStruct((n, d), jnp.float32),
    )(x, quantized)

    return (quantized_st, quant2[0, 0], util2[0, 0], compact2[0, 0], idx)
